# baseline (speedup 1.0000x reference)
import functools

import jax
import jax.numpy as jnp
from jax import lax
from jax.experimental import pallas as pl
from jax.experimental.pallas import tpu as pltpu
from jax.experimental.pallas import tpu_sc as plsc

_B, _S, _EMB = 4, 2048, 768
_N = _B * _S
_NC, _NS = 2, 16
_NW = _NC * _NS
_SPW = _S // _NW
_CH = 32
_NCH = _B * (_SPW // _CH)
_LANES = 16
_VECS = _EMB // _LANES

_mesh = plsc.VectorSubcoreMesh(core_axis_name="c", subcore_axis_name="s")


@functools.partial(
    pl.kernel,
    mesh=_mesh,
    out_type=jax.ShapeDtypeStruct((_N, _EMB), jnp.float32),
    scratch_types=[
        pltpu.VMEM((_B, _SPW), jnp.int32),
        pltpu.VMEM((_SPW, _EMB), jnp.float32),
        pltpu.VMEM((_CH, _EMB), jnp.float32),
        pltpu.VMEM((_CH, _EMB), jnp.float32),
        pltpu.SemaphoreType.DMA,
        pltpu.SemaphoreType.DMA,
        pltpu.SemaphoreType.DMA,
        pltpu.SemaphoreType.DMA,
    ],
)
def _embed(idx_hbm, tok_hbm, pos_hbm, out_hbm, idx_v, pbuf, buf0, buf1,
           sem_g0, sem_g1, sem_o0, sem_o1):
    wid = lax.axis_index("s") * _NC + lax.axis_index("c")
    bufs = (buf0, buf1)
    sems_g = (sem_g0, sem_g1)
    sems_o = (sem_o0, sem_o1)

    pltpu.sync_copy(idx_hbm.at[wid], idx_v)
    pltpu.sync_copy(pos_hbm.at[pl.ds(wid * _SPW, _SPW)], pbuf)

    def gather(ch, bb):
        b, h = divmod(ch, _SPW // _CH)
        return pltpu.async_copy(
            tok_hbm.at[idx_v.at[b, pl.ds(h * _CH, _CH)]], bufs[bb], sems_g[bb])

    g_pending = gather(0, 0)
    o_pending = [None, None]
    for ch in range(_NCH):
        bb = ch & 1
        b, h = divmod(ch, _SPW // _CH)
        row0 = b * _S + wid * _SPW + h * _CH

        g_pending.wait()
        if ch + 1 < _NCH:
            if o_pending[bb ^ 1] is not None:
                o_pending[bb ^ 1].wait()
            g_pending = gather(ch + 1, bb ^ 1)

        def add_row(r, carry, bb=bb, h=h):
            for c in range(_VECS):
                sl = pl.ds(c * _LANES, _LANES)
                bufs[bb][r, sl] = bufs[bb][r, sl] + pbuf[h * _CH + r, sl]
            return carry

        lax.fori_loop(0, _CH, add_row, 0)
        o_pending[bb] = pltpu.async_copy(
            bufs[bb], out_hbm.at[pl.ds(row0, _CH)], sems_o[bb])

    o_pending[0].wait()
    o_pending[1].wait()


def kernel(in_idx, tok_table, pos_table):
    idx = in_idx.reshape(_B, _NW, _SPW).transpose(1, 0, 2).astype(jnp.int32)
    out = _embed(idx, tok_table, pos_table)
    return out.reshape(_B, _S, _EMB)


# speedup vs baseline: 1.1672x; 1.1672x over previous
import functools

import jax
import jax.numpy as jnp
from jax import lax
from jax.experimental import pallas as pl
from jax.experimental.pallas import tpu as pltpu
from jax.experimental.pallas import tpu_sc as plsc

_B, _S, _EMB = 4, 2048, 768
_N = _B * _S
_NC, _NS = 2, 16
_NW = _NC * _NS
_SPW = _S // _NW
_CH = 32
_NCH = _B * (_SPW // _CH)
_LANES = 16
_VECS = _EMB // _LANES

_mesh = plsc.VectorSubcoreMesh(core_axis_name="c", subcore_axis_name="s")


@functools.partial(
    pl.kernel,
    mesh=_mesh,
    out_type=jax.ShapeDtypeStruct((_N, _EMB), jnp.float32),
    scratch_types=[
        pltpu.VMEM((_B, _SPW), jnp.int32),
        pltpu.VMEM((_SPW, _EMB), jnp.float32),
        pltpu.VMEM((_CH, _EMB), jnp.float32),
        pltpu.VMEM((_CH, _EMB), jnp.float32),
        pltpu.SemaphoreType.DMA,
        pltpu.SemaphoreType.DMA,
        pltpu.SemaphoreType.DMA,
        pltpu.SemaphoreType.DMA,
    ],
)
def _embed(idx_hbm, tok_hbm, pos_hbm, out_hbm, idx_v, pbuf, buf0, buf1,
           sem_g0, sem_g1, sem_o0, sem_o1):
    wid = lax.axis_index("s") * _NC + lax.axis_index("c")
    bufs = (buf0, buf1)
    sems_g = (sem_g0, sem_g1)
    sems_o = (sem_o0, sem_o1)

    pltpu.sync_copy(idx_hbm.at[wid], idx_v)
    pltpu.sync_copy(pos_hbm.at[pl.ds(wid * _SPW, _SPW)], pbuf)

    def gather(ch, bb):
        b, h = divmod(ch, _SPW // _CH)
        return pltpu.async_copy(
            tok_hbm.at[idx_v.at[b, pl.ds(h * _CH, _CH)]], bufs[bb], sems_g[bb])

    g_pending = gather(0, 0)
    o_pending = [None, None]
    for ch in range(_NCH):
        bb = ch & 1
        b, h = divmod(ch, _SPW // _CH)
        row0 = b * _S + wid * _SPW + h * _CH

        g_pending.wait()
        if ch + 1 < _NCH:
            if o_pending[bb ^ 1] is not None:
                o_pending[bb ^ 1].wait()
            g_pending = gather(ch + 1, bb ^ 1)

        def add_row(r, bb=bb, h=h):
            for c in range(_VECS):
                sl = pl.ds(c * _LANES, _LANES)
                plsc.addupdate(bufs[bb].at[r, sl], pbuf[h * _CH + r, sl])

        plsc.parallel_loop(0, _CH, 1, unroll=4)(add_row)
        o_pending[bb] = pltpu.async_copy(
            bufs[bb], out_hbm.at[pl.ds(row0, _CH)], sems_o[bb])

    o_pending[0].wait()
    o_pending[1].wait()


def kernel(in_idx, tok_table, pos_table):
    idx = in_idx.reshape(_B, _NW, _SPW).transpose(1, 0, 2).astype(jnp.int32)
    out = _embed(idx, tok_table, pos_table)
    return out.reshape(_B, _S, _EMB)


# speedup vs baseline: 1.4496x; 1.2420x over previous
import functools

import jax
import jax.numpy as jnp
from jax import lax
from jax.experimental import pallas as pl
from jax.experimental.pallas import tpu as pltpu
from jax.experimental.pallas import tpu_sc as plsc

_B, _S, _EMB = 4, 2048, 768
_N = _B * _S
_NC, _NS = 2, 16
_NW = _NC * _NS
_SPW = _S // _NW
_CH = 16
_NPH = _SPW // _CH
_LANES = 16
_VECS = _EMB // _LANES

_mesh = plsc.VectorSubcoreMesh(core_axis_name="c", subcore_axis_name="s")


@functools.partial(
    pl.kernel,
    mesh=_mesh,
    out_type=jax.ShapeDtypeStruct((_N, _EMB), jnp.float32),
    scratch_types=[
        pltpu.VMEM((_B, _SPW), jnp.int32),
        pltpu.VMEM((2, _CH, _EMB), jnp.float32),
        pltpu.VMEM((_B * _CH, _EMB), jnp.float32),
        pltpu.VMEM((_B * _CH, _EMB), jnp.float32),
        pltpu.SemaphoreType.DMA,
        pltpu.SemaphoreType.DMA,
        pltpu.SemaphoreType.DMA,
        pltpu.SemaphoreType.DMA,
        pltpu.SemaphoreType.DMA,
        pltpu.SemaphoreType.DMA,
    ],
)
def _embed(idx_hbm, tok_hbm, pos_hbm, out_hbm, idx_v, pbuf, buf0, buf1,
           sem_p0, sem_p1, sem_g0, sem_g1, sem_o0, sem_o1):
    wid = lax.axis_index("s") * _NC + lax.axis_index("c")
    bufs = (buf0, buf1)
    sems_p = (sem_p0, sem_p1)
    sems_g = (sem_g0, sem_g1)
    sems_o = (sem_o0, sem_o1)

    pltpu.sync_copy(idx_hbm.at[wid], idx_v)

    def pos_load(p):
        pp = p & 1
        return pltpu.async_copy(
            pos_hbm.at[pl.ds(wid * _SPW + p * _CH, _CH)],
            pbuf.at[pp], sems_p[pp])

    def gathers(p):
        pp = p & 1
        return [
            pltpu.async_copy(
                tok_hbm.at[idx_v.at[b, pl.ds(p * _CH, _CH)]],
                bufs[pp].at[pl.ds(b * _CH, _CH)], sems_g[pp])
            for b in range(_B)
        ]

    def outs(p):
        pp = p & 1
        return [
            pltpu.async_copy(
                bufs[pp].at[pl.ds(b * _CH, _CH)],
                out_hbm.at[pl.ds(b * _S + wid * _SPW + p * _CH, _CH)],
                sems_o[pp])
            for b in range(_B)
        ]

    p_pending = [None, None]
    g_pending = [None, None]
    o_pending = [None, None]

    p_pending[0] = pos_load(0)
    g_pending[0] = gathers(0)
    p_pending[1] = pos_load(1)
    g_pending[1] = gathers(1)

    for p in range(_NPH):
        pp = p & 1
        p_pending[pp].wait()
        for h in g_pending[pp]:
            h.wait()

        def add_rows(r, pp=pp):
            pv = [pbuf[pp, r, pl.ds(c * _LANES, _LANES)] for c in range(_VECS)]
            for b in range(_B):
                for c in range(_VECS):
                    plsc.addupdate(
                        bufs[pp].at[b * _CH + r, pl.ds(c * _LANES, _LANES)],
                        pv[c])

        plsc.parallel_loop(0, _CH, 1)(add_rows)

        o_pending[pp] = outs(p)
        if p + 2 < _NPH:
            for h in o_pending[pp]:
                h.wait()
            o_pending[pp] = None
            p_pending[pp] = pos_load(p + 2)
            g_pending[pp] = gathers(p + 2)

    for pp in range(2):
        if o_pending[pp] is not None:
            for h in o_pending[pp]:
                h.wait()


def kernel(in_idx, tok_table, pos_table):
    idx = in_idx.reshape(_B, _NW, _SPW).transpose(1, 0, 2).astype(jnp.int32)
    out = _embed(idx, tok_table, pos_table)
    return out.reshape(_B, _S, _EMB)


# speedup vs baseline: 1.4877x; 1.0263x over previous
import functools

import jax
import jax.numpy as jnp
from jax import lax
from jax.experimental import pallas as pl
from jax.experimental.pallas import tpu as pltpu
from jax.experimental.pallas import tpu_sc as plsc

_B, _S, _EMB = 4, 2048, 768
_N = _B * _S
_NC, _NS = 2, 16
_NW = _NC * _NS
_SPW = _S // _NW
_CH = 16
_NPH = _SPW // _CH
_LANES = 16
_VECS = _EMB // _LANES

_mesh = plsc.VectorSubcoreMesh(core_axis_name="c", subcore_axis_name="s")


@functools.partial(
    pl.kernel,
    mesh=_mesh,
    out_type=jax.ShapeDtypeStruct((_N, _EMB), jnp.float32),
    scratch_types=[
        pltpu.VMEM((_B, _SPW), jnp.int32),
        pltpu.VMEM((2, _CH, _EMB), jnp.float32),
        pltpu.VMEM((_B * _CH, _EMB), jnp.float32),
        pltpu.VMEM((_B * _CH, _EMB), jnp.float32),
        pltpu.SemaphoreType.DMA,
        pltpu.SemaphoreType.DMA,
        pltpu.SemaphoreType.DMA,
        pltpu.SemaphoreType.DMA,
        pltpu.SemaphoreType.DMA,
        pltpu.SemaphoreType.DMA,
        pltpu.SemaphoreType.DMA,
    ],
)
def _embed(idx_hbm, tok_hbm, pos_hbm, out_hbm, idx_v, pbuf, buf0, buf1,
           sem_p0, sem_p1, sem_g0, sem_g1, sem_o0, sem_o1, sem_i):
    wid = lax.axis_index("s") * _NC + lax.axis_index("c")
    bufs = (buf0, buf1)
    sems_p = (sem_p0, sem_p1)
    sems_g = (sem_g0, sem_g1)
    sems_o = (sem_o0, sem_o1)

    idx_copy = pltpu.async_copy(idx_hbm.at[wid], idx_v, sem_i)

    def pos_load(p):
        pp = p & 1
        return pltpu.async_copy(
            pos_hbm.at[pl.ds(wid * _SPW + p * _CH, _CH)],
            pbuf.at[pp], sems_p[pp])

    def gather_one(p, b):
        pp = p & 1
        return pltpu.async_copy(
            tok_hbm.at[idx_v.at[b, pl.ds(p * _CH, _CH)]],
            bufs[pp].at[pl.ds(b * _CH, _CH)], sems_g[pp])

    def gathers(p):
        return [gather_one(p, b) for b in range(_B)]

    def outs(p):
        pp = p & 1
        return [
            pltpu.async_copy(
                bufs[pp].at[pl.ds(b * _CH, _CH)],
                out_hbm.at[pl.ds(b * _S + wid * _SPW + p * _CH, _CH)],
                sems_o[pp])
            for b in range(_B)
        ]

    p_pending = [None, None]
    g_pending = [None, None]
    o_pending = [None, None]

    p_pending[0] = pos_load(0)
    p_pending[1] = pos_load(1)
    idx_copy.wait()
    g_pending[0] = gathers(0)
    g_pending[1] = gathers(1)

    for p in range(_NPH):
        pp = p & 1
        p_pending[pp].wait()
        for h in g_pending[pp]:
            h.wait()

        def add_rows(r, pp=pp):
            pv = [pbuf[pp, r, pl.ds(c * _LANES, _LANES)] for c in range(_VECS)]
            for b in range(_B):
                for c in range(_VECS):
                    plsc.addupdate(
                        bufs[pp].at[b * _CH + r, pl.ds(c * _LANES, _LANES)],
                        pv[c])

        plsc.parallel_loop(0, _CH, 1)(add_rows)

        o_pending[pp] = outs(p)
        if p + 2 < _NPH:
            p_pending[pp] = pos_load(p + 2)
            nxt = []
            for b in range(_B):
                o_pending[pp][b].wait()
                nxt.append(gather_one(p + 2, b))
            o_pending[pp] = None
            g_pending[pp] = nxt

    for pp in range(2):
        if o_pending[pp] is not None:
            for h in o_pending[pp]:
                h.wait()


def kernel(in_idx, tok_table, pos_table):
    idx = in_idx.reshape(_B, _NW, _SPW).transpose(1, 0, 2).astype(jnp.int32)
    out = _embed(idx, tok_table, pos_table)
    return out.reshape(_B, _S, _EMB)
